# baseline (speedup 1.0000x reference)
import functools

import jax
import jax.numpy as jnp
from jax import lax
from jax.experimental import pallas as pl
from jax.experimental.pallas import tpu as pltpu
from jax.experimental.pallas import tpu_sc as plsc

_DIM = 32
_NC = 2
_NS = 16
_NW = _NC * _NS
_CHUNK = 128
_L = 16


@functools.lru_cache
def _build(S, B, V):
    assert B == _NW * _CHUNK and S % 2 == 0
    mesh = plsc.VectorSubcoreMesh(
        core_axis_name="c", subcore_axis_name="s",
        num_cores=_NC, num_subcores=_NS)

    @functools.partial(
        pl.kernel,
        mesh=mesh,
        out_type=jax.ShapeDtypeStruct((S, _DIM, B), jnp.float32),
        scratch_types=[
            pltpu.VMEM((8, _CHUNK), jnp.int32),
            pltpu.VMEM((8, _CHUNK), jnp.int32),
            pltpu.VMEM((_CHUNK, _DIM), jnp.float32),
            pltpu.VMEM((_CHUNK, _DIM), jnp.float32),
            pltpu.VMEM((_DIM, _CHUNK), jnp.float32),
            pltpu.VMEM((_DIM, _CHUNK), jnp.float32),
            pltpu.SemaphoreType.DMA,
            pltpu.SemaphoreType.DMA,
        ],
        compiler_params=pltpu.CompilerParams(
            use_tc_tiling_on_sc=False, needs_layout_passes=False),
    )
    def emb(idx_hbm, table_hbm, out_hbm, i0, i1, g0, g1, o0, o1,
            gsem0, gsem1):
        ibuf = (i0, i1)
        gbuf = (g0, g1)
        obuf = (o0, o1)
        gsems = (gsem0, gsem1)
        wid = lax.axis_index("s") * _NC + lax.axis_index("c")
        blk = lax.div(wid, 8)
        sub = lax.rem(wid, 8)
        clo = lax.iota(jnp.int32, _L)
        chi = clo + _L

        def load_idx(s, b):
            pltpu.sync_copy(idx_hbm.at[s, blk], ibuf[b])

        def cp(b):
            return pltpu.make_async_copy(
                table_hbm.at[ibuf[b].at[sub]], gbuf[b], gsems[b])

        def transpose(b):
            g, o = gbuf[b], obuf[b]

            @pl.loop(0, _CHUNK // _L)
            def _(gi):
                for l in range(_L):
                    i = gi * _L + l
                    iv = jnp.full((_L,), i, jnp.int32)
                    plsc.store_scatter(o, [clo, iv], g[i, 0:_L])
                    plsc.store_scatter(o, [chi, iv], g[i, _L:_DIM])

        def write(s, b):
            pltpu.sync_copy(
                obuf[b], out_hbm.at[s, :, pl.ds(wid * _CHUNK, _CHUNK)])

        for b in range(2):
            load_idx(b, b)
            cp(b).start()

        @pl.loop(0, S - 2, step=2)
        def _(s):
            for b in range(2):
                cp(b).wait()
                transpose(b)
                write(s + b, b)
                load_idx(s + b + 2, b)
                cp(b).start()

        for b in range(2):
            cp(b).wait()
            transpose(b)
            write(S - 2 + b, b)

    return emb


def kernel(x, weight):
    orig_shape = x.shape
    v, dim = weight.shape
    s = x.shape[-1]
    b = x.size // s
    xt = x.T.astype(jnp.int32).reshape(s, b // _CHUNK // 8, 8, _CHUNK)
    out_phys = _build(s, b, v)(xt, weight)
    return jnp.transpose(out_phys, (2, 0, 1)).reshape(orig_shape + (dim,))
